# baseline (speedup 1.0000x reference)
import dataclasses
import functools

import jax
import jax.numpy as jnp
from jax import lax
from jax.experimental import pallas as pl
from jax.experimental.pallas import tpu as pltpu
from jax.experimental.pallas import tpu_sc as plsc

T = 128
D = 1024
F = 4096
E = 8
FB = 1024

_NC = 2
_NL = 16
_NCHUNK = T // _NL


def _logits_kernel(x_ref, wg_ref, out_ref):
    out_ref[...] = jnp.dot(x_ref[...], wg_ref[...],
                           preferred_element_type=jnp.float32)


def _sc_router(lt_hbm, gates_hbm, lslab, gslab):
    wid = lax.axis_index("s") * _NC + lax.axis_index("c")

    @pl.when(wid < _NCHUNK)
    def _():
        base = wid * (_NL * E)
        pltpu.sync_copy(lt_hbm.at[pl.ds(base, _NL * E)], lslab)
        iota = lax.iota(jnp.int32, _NL)
        ls = [plsc.load_gather(lslab, [iota * E + e]) for e in range(E)]
        m1 = ls[0]
        for e in range(1, E):
            m1 = jnp.maximum(m1, ls[e])
        found = m1 < m1
        is1 = []
        for e in range(E):
            hit = jnp.logical_and(ls[e] == m1, jnp.logical_not(found))
            is1.append(hit)
            found = jnp.logical_or(found, hit)
        l2 = [jnp.where(is1[e], -1e30, ls[e]) for e in range(E)]
        m2 = l2[0]
        for e in range(1, E):
            m2 = jnp.maximum(m2, l2[e])
        found2 = m2 < m2
        is2 = []
        for e in range(E):
            hit = jnp.logical_and(l2[e] == m2, jnp.logical_not(found2))
            is2.append(hit)
            found2 = jnp.logical_or(found2, hit)
        g1 = 1.0 / (1.0 + jnp.exp(m2 - m1))
        g2 = 1.0 - g1
        for e in range(E):
            val = jnp.where(is1[e], g1, jnp.where(is2[e], g2, 0.0))
            plsc.store_scatter(gslab, [iota * E + e], val)
        pltpu.sync_copy(gslab, gates_hbm.at[pl.ds(base, _NL * E)])


def _moe_kernel(x_ref, gates_ref, w1_ref, w3_ref, w2_ref, out_ref):
    e = pl.program_id(0)
    f = pl.program_id(1)

    @pl.when(jnp.logical_and(e == 0, f == 0))
    def _init():
        out_ref[...] = jnp.zeros_like(out_ref)

    x = x_ref[...].astype(jnp.bfloat16)
    h1 = jnp.dot(x, w1_ref[0].astype(jnp.bfloat16),
                 preferred_element_type=jnp.float32)
    h3 = jnp.dot(x, w3_ref[0].astype(jnp.bfloat16),
                 preferred_element_type=jnp.float32)
    h = (h1 * jax.lax.logistic(h1)) * h3
    iota = jax.lax.broadcasted_iota(jnp.int32, (T, E), 1)
    g = jnp.sum(jnp.where(iota == e, gates_ref[...], 0.0), axis=1,
                keepdims=True)
    out_ref[...] += jnp.dot((h * g).astype(jnp.bfloat16),
                            w2_ref[0].astype(jnp.bfloat16),
                            preferred_element_type=jnp.float32)


def kernel(hidden_states, Wg, W1, W3, W2):
    x = hidden_states.reshape(-1, hidden_states.shape[-1])

    logits = pl.pallas_call(
        _logits_kernel,
        out_shape=jax.ShapeDtypeStruct((T, E), jnp.float32),
    )(x, Wg)

    mesh = plsc.VectorSubcoreMesh(core_axis_name="c", subcore_axis_name="s")
    cp = pltpu.CompilerParams()
    if "needs_layout_passes" in pltpu.CompilerParams.__dataclass_fields__:
        cp = dataclasses.replace(cp, needs_layout_passes=False)
    router = functools.partial(
        pl.kernel, mesh=mesh,
        out_type=jax.ShapeDtypeStruct((T * E,), jnp.float32),
        scratch_types=[pltpu.VMEM((_NL * E,), jnp.float32),
                       pltpu.VMEM((_NL * E,), jnp.float32)],
        compiler_params=cp,
    )(_sc_router)
    gates = router(logits.reshape(T * E)).reshape(T, E)

    nf = F // FB
    return pl.pallas_call(
        _moe_kernel,
        grid=(E, nf),
        in_specs=[
            pl.BlockSpec((T, D), lambda e, f: (0, 0)),
            pl.BlockSpec((T, E), lambda e, f: (0, 0)),
            pl.BlockSpec((1, D, FB), lambda e, f: (e, 0, f)),
            pl.BlockSpec((1, D, FB), lambda e, f: (e, 0, f)),
            pl.BlockSpec((1, FB, D), lambda e, f: (e, f, 0)),
        ],
        out_specs=pl.BlockSpec((T, D), lambda e, f: (0, 0)),
        out_shape=jax.ShapeDtypeStruct((T, D), jnp.float32),
    )(x, gates, W1, W3, W2)


# speedup vs baseline: 1.1390x; 1.1390x over previous
import dataclasses
import functools

import jax
import jax.numpy as jnp
from jax import lax
from jax.experimental import pallas as pl
from jax.experimental.pallas import tpu as pltpu
from jax.experimental.pallas import tpu_sc as plsc

T = 128
D = 1024
F = 4096
E = 8
FB = 1024

_NC = 2
_NL = 16
_NCHUNK = T // _NL


def _logits_kernel(x_ref, wg_ref, out_ref):
    out_ref[...] = jnp.dot(x_ref[...], wg_ref[...],
                           preferred_element_type=jnp.float32)


def _sc_router(lt_hbm, gates_hbm, lslab, gslab):
    wid = lax.axis_index("s") * _NC + lax.axis_index("c")

    @pl.when(wid < _NCHUNK)
    def _():
        base = wid * (_NL * E)
        pltpu.sync_copy(lt_hbm.at[pl.ds(base, _NL * E)], lslab)
        iota = lax.iota(jnp.int32, _NL)
        ls = [plsc.load_gather(lslab, [iota * E + e]) for e in range(E)]
        m1 = ls[0]
        for e in range(1, E):
            m1 = jnp.maximum(m1, ls[e])
        found = m1 < m1
        is1 = []
        for e in range(E):
            hit = jnp.logical_and(ls[e] == m1, jnp.logical_not(found))
            is1.append(hit)
            found = jnp.logical_or(found, hit)
        l2 = [jnp.where(is1[e], -1e30, ls[e]) for e in range(E)]
        m2 = l2[0]
        for e in range(1, E):
            m2 = jnp.maximum(m2, l2[e])
        found2 = m2 < m2
        is2 = []
        for e in range(E):
            hit = jnp.logical_and(l2[e] == m2, jnp.logical_not(found2))
            is2.append(hit)
            found2 = jnp.logical_or(found2, hit)
        g1 = 1.0 / (1.0 + jnp.exp(m2 - m1))
        g2 = 1.0 - g1
        for e in range(E):
            val = jnp.where(is1[e], g1, jnp.where(is2[e], g2, 0.0))
            plsc.store_scatter(gslab, [iota * E + e], val)
        pltpu.sync_copy(gslab, gates_hbm.at[pl.ds(base, _NL * E)])


def _moe_kernel(x_ref, gates_ref, w1_ref, w3_ref, w2_ref, out_ref):
    e = pl.program_id(0)
    f = pl.program_id(1)

    @pl.when(jnp.logical_and(e == 0, f == 0))
    def _init():
        out_ref[...] = jnp.zeros_like(out_ref)

    x = x_ref[...].astype(jnp.bfloat16)
    h1 = jnp.dot(x, w1_ref[0].astype(jnp.bfloat16),
                 preferred_element_type=jnp.float32)
    h3 = jnp.dot(x, w3_ref[0].astype(jnp.bfloat16),
                 preferred_element_type=jnp.float32)
    h = (h1 * jax.lax.logistic(h1)) * h3
    iota = jax.lax.broadcasted_iota(jnp.int32, (T, E), 1)
    g = jnp.sum(jnp.where(iota == e, gates_ref[...], 0.0), axis=1,
                keepdims=True)
    out_ref[...] += jnp.dot((h * g).astype(jnp.bfloat16),
                            w2_ref[0].astype(jnp.bfloat16),
                            preferred_element_type=jnp.float32)


def kernel(hidden_states, Wg, W1, W3, W2):
    x = hidden_states.reshape(-1, hidden_states.shape[-1])

    logits = pl.pallas_call(
        _logits_kernel,
        out_shape=jax.ShapeDtypeStruct((T, E), jnp.float32),
    )(x, Wg)

    def _tc_router(lt_ref, g_ref):
        p = lt_ref[...]
        iota2 = jax.lax.broadcasted_iota(jnp.int32, (T, E), 1)
        v1 = jnp.max(p, axis=1, keepdims=True)
        i1 = jnp.min(jnp.where(p == v1, iota2, E), axis=1, keepdims=True)
        mask1 = iota2 == i1
        p2 = jnp.where(mask1, -1e30, p)
        v2 = jnp.max(p2, axis=1, keepdims=True)
        i2 = jnp.min(jnp.where(p2 == v2, iota2, E), axis=1, keepdims=True)
        mask2 = iota2 == i2
        g1 = 1.0 / (1.0 + jnp.exp(v2 - v1))
        g_ref[...] = jnp.where(mask1, g1, 0.0) + jnp.where(mask2, 1.0 - g1, 0.0)

    gates = pl.pallas_call(
        _tc_router,
        out_shape=jax.ShapeDtypeStruct((T, E), jnp.float32),
    )(logits)

    nf = F // FB
    return pl.pallas_call(
        _moe_kernel,
        grid=(E, nf),
        in_specs=[
            pl.BlockSpec((T, D), lambda e, f: (0, 0)),
            pl.BlockSpec((T, E), lambda e, f: (0, 0)),
            pl.BlockSpec((1, D, FB), lambda e, f: (e, 0, f)),
            pl.BlockSpec((1, D, FB), lambda e, f: (e, 0, f)),
            pl.BlockSpec((1, FB, D), lambda e, f: (e, f, 0)),
        ],
        out_specs=pl.BlockSpec((T, D), lambda e, f: (0, 0)),
        out_shape=jax.ShapeDtypeStruct((T, D), jnp.float32),
    )(x, gates, W1, W3, W2)
